# Initial kernel scaffold; baseline (speedup 1.0000x reference)
import jax
import jax.numpy as jnp
from jax.experimental import pallas as pl


def kernel(x0, x1, x2, x3, ei0, ei1, ei2, ei3, vals0, vals1, vals2, vals3, Wb0, Wb1, Wb2, Wb3, Wo0, Wo1, Wo2, Wo3, Wf0, Wf1, Wf2, Wf3, Wc0_0, Wc0_1, Wc0_2, Wc0_3, Wc1, Wc2, Gw, Gb):
    raise NotImplementedError("write your pallas kernel here")


# speedup vs baseline: 1.9612x; 1.9612x over previous
import functools

import jax
import jax.numpy as jnp
from jax.experimental import pallas as pl
from jax.experimental.pallas import tpu as pltpu

_N = 22325
_E = 357200
_BLK = 512
_NP = 22528


def _mm_body(x_ref, w_ref, o_ref, *, relu_in):
    x = x_ref[...]
    if relu_in:
        x = jnp.maximum(x, 0.0)
    o_ref[...] = jnp.dot(x, w_ref[...], preferred_element_type=jnp.float32)


def _mm(x, w, relu_in=False):
    k = x.shape[1]
    m = w.shape[1]
    return pl.pallas_call(
        functools.partial(_mm_body, relu_in=relu_in),
        grid=(_NP // _BLK,),
        in_specs=[
            pl.BlockSpec((_BLK, k), lambda i: (i, 0)),
            pl.BlockSpec((k, m), lambda i: (0, 0)),
        ],
        out_specs=pl.BlockSpec((_BLK, m), lambda i: (i, 0)),
        out_shape=jax.ShapeDtypeStruct((_NP, m), jnp.float32),
    )(x, w)


def _spmm(ei, vals, x):
    msg = vals[:, None] * jnp.take(x[:_N], ei[0], axis=0)
    out = jax.ops.segment_sum(msg, ei[1], num_segments=_N)
    return jnp.pad(out, ((0, _NP - _N), (0, 0)))


def _gate_body(z0_ref, z1_ref, z2_ref, z3_ref, wg_ref, b_ref,
               o0_ref, o1_ref, o2_ref, o3_ref):
    zs = [z0_ref[...], z1_ref[...], z2_ref[...], z3_ref[...]]
    zcat = jnp.concatenate(zs, axis=1)
    s = jnp.dot(zcat, wg_ref[...], preferred_element_type=jnp.float32)
    s = jnp.maximum(s + b_ref[...], 0.0)
    outs = [o0_ref, o1_ref, o2_ref, o3_ref]
    for g in range(4):
        sg = s[:, 4 * g:4 * g + 4]
        if g == 0:
            mx = jnp.max(sg, axis=1, keepdims=True)
            e = jnp.exp(sg - mx)
            sg = e / jnp.sum(e, axis=1, keepdims=True)
        sg = jnp.tanh(sg)
        acc = sg[:, 0:1] * zs[0]
        for i in range(1, 4):
            acc = acc + sg[:, i:i + 1] * zs[i]
        outs[g][...] = acc


def _gate(z, wg, b16):
    spec_z = pl.BlockSpec((_BLK, 128), lambda i: (i, 0))
    out_sh = jax.ShapeDtypeStruct((_NP, 128), jnp.float32)
    return pl.pallas_call(
        _gate_body,
        grid=(_NP // _BLK,),
        in_specs=[spec_z] * 4 + [
            pl.BlockSpec((512, 16), lambda i: (0, 0)),
            pl.BlockSpec((1, 16), lambda i: (0, 0)),
        ],
        out_specs=[spec_z] * 4,
        out_shape=[out_sh] * 4,
    )(z[0], z[1], z[2], z[3], wg, b16)


def kernel(x0, x1, x2, x3, ei0, ei1, ei2, ei3, vals0, vals1, vals2, vals3,
           Wb0, Wb1, Wb2, Wb3, Wo0, Wo1, Wo2, Wo3, Wf0, Wf1, Wf2, Wf3,
           Wc0_0, Wc0_1, Wc0_2, Wc0_3, Wc1, Wc2, Gw, Gb):
    xs = [x0, x1, x2, x3]
    eis = [ei0, ei1, ei2, ei3]
    vs = [vals0, vals1, vals2, vals3]
    Wbs = [Wb0, Wb1, Wb2, Wb3]
    Wos = [Wo0, Wo1, Wo2, Wo3]
    Wfs = [Wf0, Wf1, Wf2, Wf3]
    Wc0s = [Wc0_0, Wc0_1, Wc0_2, Wc0_3]

    def blockdiag(a, b):
        k, m = a.shape
        top = jnp.concatenate([a, jnp.zeros((k, b.shape[1]), a.dtype)], axis=1)
        bot = jnp.concatenate([jnp.zeros((b.shape[0], m), a.dtype), b], axis=1)
        return jnp.concatenate([top, bot], axis=0)

    z = []
    for v in range(4):
        x = jnp.pad(xs[v], ((0, _NP - _N), (0, 0)))
        w1 = jnp.concatenate([Wbs[v], Wc0s[v]], axis=1)
        y1 = _spmm(eis[v], vs[v], _mm(x, w1))
        w2 = blockdiag(Wos[v], Wc1)
        y2 = _spmm(eis[v], vs[v], _mm(y1, w2, relu_in=True))
        w3 = blockdiag(Wfs[v], Wc2)
        y3 = _spmm(eis[v], vs[v], _mm(y2, w3, relu_in=True))
        h = jnp.maximum(y1[:, :64], 0.0)
        m = jnp.maximum(y2[:, :64], 0.0)
        f = y3[:, :64]
        zv = (h + m + f) / 3.0
        c = jnp.maximum(y3[:, 64:], 0.0)
        z.append(jnp.concatenate([c, zv], axis=1))

    wg = jnp.zeros((512, 16), jnp.float32)
    for g in range(4):
        for i in range(4):
            wg = wg.at[128 * i:128 * (i + 1), 4 * g + i].set(Gw[g, i])
    b16 = Gb.reshape(1, 16)

    zg = _gate(z, wg, b16)
    out = tuple(a[:_N] for a in zg) + tuple(a[:_N] for a in z)
    return out


# speedup vs baseline: 5.7406x; 2.9271x over previous
import functools

import jax
import jax.numpy as jnp
from jax import lax
from jax.experimental import pallas as pl
from jax.experimental.pallas import tpu as pltpu
from jax.experimental.pallas import tpu_sc as plsc

_N = 22325
_E = 357200
_BLK = 512
_NPAD = 22528
_HALF = 11264
_TRASH = _HALF
_NACC = _HALF + 128
_SLAB = 704
_CHUNK = 128
_NB = 8
_NBLOCKS = 22
_EROWS = _NB * _NBLOCKS
_EPT = _EROWS * _CHUNK
_EP = 16 * _EPT


def _sc_spmm_body(src_hbm, dst_hbm, vals_hbm, x_hbm, out_hbm,
                  src_v, dst_v, vals_v, rows0, rows1, didx, acc,
                  sem0, sem1):
    c = lax.axis_index("c")
    s = lax.axis_index("s")
    rowsb = (rows0, rows1)
    sems = (sem0, sem1)
    dbase = c * _HALF

    def zloop(i, _):
        for q in range(8):
            rows0[i, pl.ds(16 * q, 16)] = jnp.zeros((16,), jnp.float32)
        return 0
    lax.fori_loop(0, _CHUNK, zloop, 0)
    for t in range(5):
        pltpu.sync_copy(rows0, acc.at[pl.ds(s * _SLAB + t * 128, 128)])
    pltpu.sync_copy(rows0.at[pl.ds(0, 64)],
                    acc.at[pl.ds(s * _SLAB + 640, 64)])
    plsc.subcore_barrier()

    def gather_issue(b, p):
        pltpu.async_copy(x_hbm.at[src_v.at[b]], rowsb[p], sems[p])

    def scale_scatter(b, p):
        pltpu.make_async_copy(x_hbm.at[src_v.at[b]], rowsb[p],
                              sems[p]).wait()

        def sloop(k, _):
            sl16 = pl.ds(16 * k, 16)
            dv = dst_v[b, sl16] - dbase
            ok = (dv >= 0) & (dv < _HALF)
            didx[sl16] = jnp.where(ok, dv, _TRASH)
            vv = vals_v[b, sl16]
            for j in range(16):
                vj = vv[j]
                e = 16 * k + j
                for q in range(8):
                    sl = pl.ds(16 * q, 16)
                    rowsb[p][e, sl] = rowsb[p][e, sl] * vj
            return 0
        lax.fori_loop(0, _CHUNK // 16, sloop, 0)
        pltpu.sync_copy(rowsb[p], acc.at[didx], add=True)

    def big_block(bb, _):
        base = s * _EROWS + bb * _NB
        pltpu.sync_copy(src_hbm.at[pl.ds(base, _NB)], src_v)
        pltpu.sync_copy(dst_hbm.at[pl.ds(base, _NB)], dst_v)
        pltpu.sync_copy(vals_hbm.at[pl.ds(base, _NB)], vals_v)
        gather_issue(0, 0)
        for b in range(1, _NB):
            gather_issue(b, b % 2)
            scale_scatter(b - 1, (b - 1) % 2)
        scale_scatter(_NB - 1, (_NB - 1) % 2)
        return 0
    lax.fori_loop(0, _NBLOCKS, big_block, 0)

    plsc.subcore_barrier()
    pltpu.sync_copy(acc.at[pl.ds(s * _SLAB, _SLAB)],
                    out_hbm.at[pl.ds(c * _HALF + s * _SLAB, _SLAB)])


def _sc_spmm(src2d, dst2d, vals2d, x):
    mesh = plsc.VectorSubcoreMesh(core_axis_name="c", subcore_axis_name="s")
    f = functools.partial(
        pl.kernel,
        out_type=jax.ShapeDtypeStruct((_NPAD, 128), jnp.float32),
        mesh=mesh,
        scratch_types=[
            pltpu.VMEM((_NB, _CHUNK), jnp.int32),
            pltpu.VMEM((_NB, _CHUNK), jnp.int32),
            pltpu.VMEM((_NB, _CHUNK), jnp.float32),
            pltpu.VMEM((_CHUNK, 128), jnp.float32),
            pltpu.VMEM((_CHUNK, 128), jnp.float32),
            pltpu.VMEM((_CHUNK,), jnp.int32),
            pltpu.VMEM_SHARED((_NACC, 128), jnp.float32),
            pltpu.SemaphoreType.DMA,
            pltpu.SemaphoreType.DMA,
        ],
    )(_sc_spmm_body)
    return f(src2d, dst2d, vals2d, x)


def _mm_body(x_ref, w_ref, o_ref, *, relu_in):
    x = x_ref[...]
    if relu_in:
        x = jnp.maximum(x, 0.0)
    o_ref[...] = jnp.dot(x, w_ref[...], preferred_element_type=jnp.float32)


def _mm(x, w, relu_in=False):
    return pl.pallas_call(
        functools.partial(_mm_body, relu_in=relu_in),
        grid=(_NPAD // _BLK,),
        in_specs=[
            pl.BlockSpec((_BLK, 128), lambda i: (i, 0)),
            pl.BlockSpec((128, 128), lambda i: (0, 0)),
        ],
        out_specs=pl.BlockSpec((_BLK, 128), lambda i: (i, 0)),
        out_shape=jax.ShapeDtypeStruct((_NPAD, 128), jnp.float32),
    )(x, w)


def _gate_body(y10, y11, y12, y13, y20, y21, y22, y23,
               y30, y31, y32, y33, wg_ref, b_ref, *out_refs):
    y1s = [y10, y11, y12, y13]
    y2s = [y20, y21, y22, y23]
    y3s = [y30, y31, y32, y33]
    zs = []
    for v in range(4):
        h = jnp.maximum(y1s[v][:, :64], 0.0)
        m = jnp.maximum(y2s[v][:, :64], 0.0)
        f = y3s[v][:, :64]
        cbr = jnp.maximum(y3s[v][:, 64:], 0.0)
        zv = (h + m + f) / 3.0
        zs.append(jnp.concatenate([cbr, zv], axis=1))
    zcat = jnp.concatenate(zs, axis=1)
    sc = jnp.dot(zcat, wg_ref[...], preferred_element_type=jnp.float32)
    sc = jnp.maximum(sc + b_ref[...], 0.0)
    for g in range(4):
        sg = sc[:, 4 * g:4 * g + 4]
        if g == 0:
            mx = jnp.max(sg, axis=1, keepdims=True)
            e = jnp.exp(sg - mx)
            sg = e / jnp.sum(e, axis=1, keepdims=True)
        sg = jnp.tanh(sg)
        acc = sg[:, 0:1] * zs[0]
        for i in range(1, 4):
            acc = acc + sg[:, i:i + 1] * zs[i]
        out_refs[g][...] = acc
    for v in range(4):
        out_refs[4 + v][...] = zs[v]


def _gate(y1s, y2s, y3s, wg, b16):
    spec = pl.BlockSpec((_BLK, 128), lambda i: (i, 0))
    out_sh = jax.ShapeDtypeStruct((_NPAD, 128), jnp.float32)
    return pl.pallas_call(
        _gate_body,
        grid=(_NPAD // _BLK,),
        in_specs=[spec] * 12 + [
            pl.BlockSpec((512, 16), lambda i: (0, 0)),
            pl.BlockSpec((1, 16), lambda i: (0, 0)),
        ],
        out_specs=[spec] * 8,
        out_shape=[out_sh] * 8,
    )(*y1s, *y2s, *y3s, wg, b16)


def kernel(x0, x1, x2, x3, ei0, ei1, ei2, ei3, vals0, vals1, vals2, vals3,
           Wb0, Wb1, Wb2, Wb3, Wo0, Wo1, Wo2, Wo3, Wf0, Wf1, Wf2, Wf3,
           Wc0_0, Wc0_1, Wc0_2, Wc0_3, Wc1, Wc2, Gw, Gb):
    xs = [x0, x1, x2, x3]
    eis = [ei0, ei1, ei2, ei3]
    vs = [vals0, vals1, vals2, vals3]
    Wbs = [Wb0, Wb1, Wb2, Wb3]
    Wos = [Wo0, Wo1, Wo2, Wo3]
    Wfs = [Wf0, Wf1, Wf2, Wf3]
    Wc0s = [Wc0_0, Wc0_1, Wc0_2, Wc0_3]

    def blockdiag(a, b):
        k, m = a.shape
        top = jnp.concatenate([a, jnp.zeros((k, b.shape[1]), a.dtype)], axis=1)
        bot = jnp.concatenate([jnp.zeros((b.shape[0], m), a.dtype), b], axis=1)
        return jnp.concatenate([top, bot], axis=0)

    y1s, y2s, y3s = [], [], []
    for v in range(4):
        src = jnp.concatenate(
            [eis[v][0], jnp.zeros((_EP - _E,), jnp.int32)]).reshape(-1, _CHUNK)
        dst = jnp.concatenate(
            [eis[v][1], jnp.zeros((_EP - _E,), jnp.int32)]).reshape(-1, _CHUNK)
        val = jnp.concatenate(
            [vs[v], jnp.zeros((_EP - _E,), jnp.float32)]).reshape(-1, _CHUNK)
        x = jnp.pad(xs[v], ((0, _NPAD - _N), (0, 0)))
        w1 = jnp.concatenate([Wbs[v], Wc0s[v]], axis=1)
        y1 = _sc_spmm(src, dst, val, _mm(x, w1))
        w2 = blockdiag(Wos[v], Wc1)
        y2 = _sc_spmm(src, dst, val, _mm(y1, w2, relu_in=True))
        w3 = blockdiag(Wfs[v], Wc2)
        y3 = _sc_spmm(src, dst, val, _mm(y2, w3, relu_in=True))
        y1s.append(y1)
        y2s.append(y2)
        y3s.append(y3)

    wg = jnp.zeros((512, 16), jnp.float32)
    for g in range(4):
        for i in range(4):
            wg = wg.at[128 * i:128 * (i + 1), 4 * g + i].set(Gw[g, i])
    b16 = Gb.reshape(1, 16)

    outs = _gate(y1s, y2s, y3s, wg, b16)
    return tuple(a[:_N] for a in outs)


# speedup vs baseline: 5.9243x; 1.0320x over previous
import functools

import jax
import jax.numpy as jnp
from jax import lax
from jax.experimental import pallas as pl
from jax.experimental.pallas import tpu as pltpu
from jax.experimental.pallas import tpu_sc as plsc

_N = 22325
_E = 357200
_BLK = 512
_NPAD = 22528
_HALF = 11264
_TRASH = _HALF
_NACC = _HALF + 128
_SLAB = 704
_CHUNK = 128
_NB = 16
_NBLOCKS = 11
_EROWS = _NB * _NBLOCKS
_EPT = _EROWS * _CHUNK
_EP = 16 * _EPT


def _sc_spmm_body(src_hbm, dst_hbm, vals_hbm, x_hbm, out_hbm,
                  src_v, dst_v, vals_v, rows0, rows1, didx0, didx1, acc,
                  sem0, sem1, ssem0, ssem1):
    c = lax.axis_index("c")
    s = lax.axis_index("s")
    rowsb = (rows0, rows1)
    didxb = (didx0, didx1)
    sems = (sem0, sem1)
    ssems = (ssem0, ssem1)
    dbase = c * _HALF

    def zloop(i, _):
        for q in range(8):
            rows0[i, pl.ds(16 * q, 16)] = jnp.zeros((16,), jnp.float32)
        return 0
    lax.fori_loop(0, _CHUNK, zloop, 0)
    for t in range(5):
        pltpu.sync_copy(rows0, acc.at[pl.ds(s * _SLAB + t * 128, 128)])
    pltpu.sync_copy(rows0.at[pl.ds(0, 64)],
                    acc.at[pl.ds(s * _SLAB + 640, 64)])
    plsc.subcore_barrier()

    def gather_issue(b, p):
        return pltpu.async_copy(x_hbm.at[src_v.at[b]], rowsb[p], sems[p])

    def scatter_issue(b, p):
        return pltpu.async_copy(rowsb[p], acc.at[didxb[p]], ssems[p],
                                add=True)

    def scale(b, p):
        def sloop(k, _):
            sl16 = pl.ds(16 * k, 16)
            dv = dst_v[b, sl16] - dbase
            ok = (dv >= 0) & (dv < _HALF)
            didxb[p][sl16] = jnp.where(ok, dv, _TRASH)
            vv = vals_v[b, sl16]
            for j in range(16):
                vj = vv[j]
                e = 16 * k + j
                for q in range(8):
                    sl = pl.ds(16 * q, 16)
                    rowsb[p][e, sl] = rowsb[p][e, sl] * vj
            return 0
        lax.fori_loop(0, _CHUNK // 16, sloop, 0)

    def big_block(bb, _):
        base = s * _EROWS + bb * _NB
        pltpu.sync_copy(src_hbm.at[pl.ds(base, _NB)], src_v)
        pltpu.sync_copy(dst_hbm.at[pl.ds(base, _NB)], dst_v)
        pltpu.sync_copy(vals_hbm.at[pl.ds(base, _NB)], vals_v)
        gdesc = [None, None]
        sdesc = [None, None]
        gdesc[0] = gather_issue(0, 0)
        for b in range(1, _NB):
            p, pp = b % 2, (b - 1) % 2
            if b >= 2:
                sdesc[p].wait()
            gdesc[p] = gather_issue(b, p)
            gdesc[pp].wait()
            scale(b - 1, pp)
            sdesc[pp] = scatter_issue(b - 1, pp)
        last = _NB - 1
        p = last % 2
        gdesc[p].wait()
        scale(last, p)
        sdesc[p] = scatter_issue(last, p)
        sdesc[1 - p].wait()
        sdesc[p].wait()
        return 0
    lax.fori_loop(0, _NBLOCKS, big_block, 0)

    plsc.subcore_barrier()
    pltpu.sync_copy(acc.at[pl.ds(s * _SLAB, _SLAB)],
                    out_hbm.at[pl.ds(c * _HALF + s * _SLAB, _SLAB)])


def _sc_spmm(src2d, dst2d, vals2d, x):
    mesh = plsc.VectorSubcoreMesh(core_axis_name="c", subcore_axis_name="s")
    f = functools.partial(
        pl.kernel,
        out_type=jax.ShapeDtypeStruct((_NPAD, 128), jnp.float32),
        mesh=mesh,
        scratch_types=[
            pltpu.VMEM((_NB, _CHUNK), jnp.int32),
            pltpu.VMEM((_NB, _CHUNK), jnp.int32),
            pltpu.VMEM((_NB, _CHUNK), jnp.float32),
            pltpu.VMEM((_CHUNK, 128), jnp.float32),
            pltpu.VMEM((_CHUNK, 128), jnp.float32),
            pltpu.VMEM((_CHUNK,), jnp.int32),
            pltpu.VMEM((_CHUNK,), jnp.int32),
            pltpu.VMEM_SHARED((_NACC, 128), jnp.float32),
            pltpu.SemaphoreType.DMA,
            pltpu.SemaphoreType.DMA,
            pltpu.SemaphoreType.DMA,
            pltpu.SemaphoreType.DMA,
        ],
    )(_sc_spmm_body)
    return f(src2d, dst2d, vals2d, x)


def _mm_body(x_ref, w_ref, o_ref, *, relu_in):
    x = x_ref[...]
    if relu_in:
        x = jnp.maximum(x, 0.0)
    o_ref[...] = jnp.dot(x, w_ref[...], preferred_element_type=jnp.float32)


def _mm(x, w, relu_in=False):
    return pl.pallas_call(
        functools.partial(_mm_body, relu_in=relu_in),
        grid=(_NPAD // _BLK,),
        in_specs=[
            pl.BlockSpec((_BLK, 128), lambda i: (i, 0)),
            pl.BlockSpec((128, 128), lambda i: (0, 0)),
        ],
        out_specs=pl.BlockSpec((_BLK, 128), lambda i: (i, 0)),
        out_shape=jax.ShapeDtypeStruct((_NPAD, 128), jnp.float32),
    )(x, w)


def _gate_body(y10, y11, y12, y13, y20, y21, y22, y23,
               y30, y31, y32, y33, wg_ref, b_ref, *out_refs):
    y1s = [y10, y11, y12, y13]
    y2s = [y20, y21, y22, y23]
    y3s = [y30, y31, y32, y33]
    zs = []
    for v in range(4):
        h = jnp.maximum(y1s[v][:, :64], 0.0)
        m = jnp.maximum(y2s[v][:, :64], 0.0)
        f = y3s[v][:, :64]
        cbr = jnp.maximum(y3s[v][:, 64:], 0.0)
        zv = (h + m + f) / 3.0
        zs.append(jnp.concatenate([cbr, zv], axis=1))
    zcat = jnp.concatenate(zs, axis=1)
    sc = jnp.dot(zcat, wg_ref[...], preferred_element_type=jnp.float32)
    sc = jnp.maximum(sc + b_ref[...], 0.0)
    for g in range(4):
        sg = sc[:, 4 * g:4 * g + 4]
        if g == 0:
            mx = jnp.max(sg, axis=1, keepdims=True)
            e = jnp.exp(sg - mx)
            sg = e / jnp.sum(e, axis=1, keepdims=True)
        sg = jnp.tanh(sg)
        acc = sg[:, 0:1] * zs[0]
        for i in range(1, 4):
            acc = acc + sg[:, i:i + 1] * zs[i]
        out_refs[g][...] = acc
    for v in range(4):
        out_refs[4 + v][...] = zs[v]


def _gate(y1s, y2s, y3s, wg, b16):
    spec = pl.BlockSpec((_BLK, 128), lambda i: (i, 0))
    out_sh = jax.ShapeDtypeStruct((_NPAD, 128), jnp.float32)
    return pl.pallas_call(
        _gate_body,
        grid=(_NPAD // _BLK,),
        in_specs=[spec] * 12 + [
            pl.BlockSpec((512, 16), lambda i: (0, 0)),
            pl.BlockSpec((1, 16), lambda i: (0, 0)),
        ],
        out_specs=[spec] * 8,
        out_shape=[out_sh] * 8,
    )(*y1s, *y2s, *y3s, wg, b16)


def kernel(x0, x1, x2, x3, ei0, ei1, ei2, ei3, vals0, vals1, vals2, vals3,
           Wb0, Wb1, Wb2, Wb3, Wo0, Wo1, Wo2, Wo3, Wf0, Wf1, Wf2, Wf3,
           Wc0_0, Wc0_1, Wc0_2, Wc0_3, Wc1, Wc2, Gw, Gb):
    xs = [x0, x1, x2, x3]
    eis = [ei0, ei1, ei2, ei3]
    vs = [vals0, vals1, vals2, vals3]
    Wbs = [Wb0, Wb1, Wb2, Wb3]
    Wos = [Wo0, Wo1, Wo2, Wo3]
    Wfs = [Wf0, Wf1, Wf2, Wf3]
    Wc0s = [Wc0_0, Wc0_1, Wc0_2, Wc0_3]

    def blockdiag(a, b):
        k, m = a.shape
        top = jnp.concatenate([a, jnp.zeros((k, b.shape[1]), a.dtype)], axis=1)
        bot = jnp.concatenate([jnp.zeros((b.shape[0], m), a.dtype), b], axis=1)
        return jnp.concatenate([top, bot], axis=0)

    y1s, y2s, y3s = [], [], []
    for v in range(4):
        src = jnp.concatenate(
            [eis[v][0], jnp.zeros((_EP - _E,), jnp.int32)]).reshape(-1, _CHUNK)
        dst = jnp.concatenate(
            [eis[v][1], jnp.zeros((_EP - _E,), jnp.int32)]).reshape(-1, _CHUNK)
        val = jnp.concatenate(
            [vs[v], jnp.zeros((_EP - _E,), jnp.float32)]).reshape(-1, _CHUNK)
        x = jnp.pad(xs[v], ((0, _NPAD - _N), (0, 0)))
        w1 = jnp.concatenate([Wbs[v], Wc0s[v]], axis=1)
        y1 = _sc_spmm(src, dst, val, _mm(x, w1))
        w2 = blockdiag(Wos[v], Wc1)
        y2 = _sc_spmm(src, dst, val, _mm(y1, w2, relu_in=True))
        w3 = blockdiag(Wfs[v], Wc2)
        y3 = _sc_spmm(src, dst, val, _mm(y2, w3, relu_in=True))
        y1s.append(y1)
        y2s.append(y2)
        y3s.append(y3)

    wg = jnp.zeros((512, 16), jnp.float32)
    for g in range(4):
        for i in range(4):
            wg = wg.at[128 * i:128 * (i + 1), 4 * g + i].set(Gw[g, i])
    b16 = Gb.reshape(1, 16)

    outs = _gate(y1s, y2s, y3s, wg, b16)
    return tuple(a[:_N] for a in outs)
